# baseline (speedup 1.0000x reference)
import jax
import jax.numpy as jnp
from jax.experimental import pallas as pl
from jax.experimental.pallas import tpu as pltpu

N_TOK = 4096
D_MODEL = 4096
N_EXP = 64
CAP = 64
T_BLK = 512
GRID = N_TOK // T_BLK
R_BLK = 512
RGRID = (N_EXP * CAP) // R_BLK


def _gate_meta(x_ref, w_ref, flat_ref, gmax_ref, stats_ref, cnt_ref, gsum_ref):
    i = pl.program_id(0)

    @pl.when(i == 0)
    def _init():
        cnt_ref[...] = jnp.zeros_like(cnt_ref)
        gsum_ref[...] = jnp.zeros_like(gsum_ref)

    x = x_ref[...]
    w = w_ref[...]
    logits = jax.lax.dot_general(
        w, x, (((1,), (1,)), ((), ())), preferred_element_type=jnp.float32
    )
    sub = jax.lax.broadcasted_iota(jnp.int32, (N_EXP, T_BLK), 0)
    m = jnp.max(logits, axis=0, keepdims=True)
    ex = jnp.exp(logits - m)
    gates = ex / jnp.sum(ex, axis=0, keepdims=True)
    gmax = jnp.max(gates, axis=0, keepdims=True)
    eidx = jnp.min(jnp.where(gates == gmax, sub, N_EXP), axis=0, keepdims=True)
    onehot = (sub == eidx).astype(jnp.float32)

    r = jax.lax.broadcasted_iota(jnp.int32, (T_BLK, T_BLK), 0)
    c = jax.lax.broadcasted_iota(jnp.int32, (T_BLK, T_BLK), 1)
    tri = (r <= c).astype(jnp.float32)
    cum = jnp.dot(onehot, tri, preferred_element_type=jnp.float32)

    prev = cnt_ref[...]
    pos = jnp.sum((cum - 1.0 + prev) * onehot, axis=0, keepdims=True)
    pos = pos.astype(jnp.int32)
    keep = pos < CAP
    flat_ref[...] = jnp.where(keep, eidx * CAP + pos, -1).reshape(1, 1, T_BLK)
    gmax_ref[...] = gmax.reshape(1, 1, T_BLK)

    cnt_ref[...] = prev + cum[:, T_BLK - 1 : T_BLK]
    gsum_ref[...] = gsum_ref[...] + jnp.sum(gates, axis=1, keepdims=True)

    @pl.when(i == GRID - 1)
    def _fin():
        cnts = cnt_ref[...]
        gs = gsum_ref[...]
        laux = jnp.sum(cnts * gs) * jnp.float32(N_EXP / (N_TOK * N_TOK))
        lane = jax.lax.broadcasted_iota(jnp.int32, (N_EXP, 8), 1)
        stats_ref[...] = jnp.where(
            lane == 0,
            jnp.broadcast_to(cnts, (N_EXP, 8)),
            jnp.where(lane == 1, jnp.broadcast_to(gs, (N_EXP, 8)), laux),
        )


def _dispatch_rows(flat_ref, gmax_ref, cw_ref, m8_ref):
    j = pl.program_id(0)
    flat = flat_ref[...]
    gmax = gmax_ref[...]
    row = jax.lax.broadcasted_iota(jnp.int32, (R_BLK, N_TOK), 0) + j * R_BLK
    hit = row == flat
    cw_ref[...] = jnp.where(hit, gmax, 0.0)
    m8_ref[...] = hit.astype(jnp.int8)


def _run_meta(x, W):
    return pl.pallas_call(
        _gate_meta,
        grid=(GRID,),
        in_specs=[
            pl.BlockSpec((T_BLK, D_MODEL), lambda i: (i, 0)),
            pl.BlockSpec((N_EXP, D_MODEL), lambda i: (0, 0)),
        ],
        out_specs=[
            pl.BlockSpec((1, 1, T_BLK), lambda i: (i, 0, 0)),
            pl.BlockSpec((1, 1, T_BLK), lambda i: (i, 0, 0)),
            pl.BlockSpec((N_EXP, 8), lambda i: (0, 0)),
        ],
        out_shape=[
            jax.ShapeDtypeStruct((GRID, 1, T_BLK), jnp.int32),
            jax.ShapeDtypeStruct((GRID, 1, T_BLK), jnp.float32),
            jax.ShapeDtypeStruct((N_EXP, 8), jnp.float32),
        ],
        scratch_shapes=[
            pltpu.VMEM((N_EXP, 1), jnp.float32),
            pltpu.VMEM((N_EXP, 1), jnp.float32),
        ],
        compiler_params=pltpu.CompilerParams(
            dimension_semantics=("arbitrary",),
        ),
    )(x, W)


def _run_dispatch(flat, gmax):
    return pl.pallas_call(
        _dispatch_rows,
        grid=(RGRID,),
        in_specs=[
            pl.BlockSpec((1, N_TOK), lambda j: (0, 0)),
            pl.BlockSpec((1, N_TOK), lambda j: (0, 0)),
        ],
        out_specs=[
            pl.BlockSpec((R_BLK, N_TOK), lambda j: (j, 0)),
            pl.BlockSpec((R_BLK, N_TOK), lambda j: (j, 0)),
        ],
        out_shape=[
            jax.ShapeDtypeStruct((N_EXP * CAP, N_TOK), jnp.float32),
            jax.ShapeDtypeStruct((N_EXP * CAP, N_TOK), jnp.int8),
        ],
        compiler_params=pltpu.CompilerParams(
            dimension_semantics=("arbitrary",),
        ),
    )(flat, gmax)


def _kernel_impl(x, W):
    flatb, gmaxb, stats = _run_meta(x, W)
    flat = flatb.reshape(1, N_TOK)
    gmax = gmaxb.reshape(1, N_TOK)
    cw2, m8 = _run_dispatch(flat, gmax)
    l_aux = stats[0, 2]
    exp_counts = stats[:N_EXP, 0].astype(jnp.int32)
    cw3 = cw2.reshape(N_EXP, CAP, N_TOK)
    m3 = m8.reshape(N_EXP, CAP, N_TOK)
    combine_weights = jnp.transpose(cw3, (2, 0, 1))
    dispatch_mask = jnp.transpose(m3, (2, 0, 1)).astype(jnp.bool_)
    return (l_aux, combine_weights, dispatch_mask, exp_counts)


def kernel(x, W):
    return jax.jit(_kernel_impl)(x, W)


# speedup vs baseline: 1.0021x; 1.0021x over previous
import jax
import jax.numpy as jnp
from jax.experimental import pallas as pl
from jax.experimental.pallas import tpu as pltpu

N_TOK = 4096
D_MODEL = 4096
N_EXP = 64
CAP = 64
T_BLK = 512
GRID = N_TOK // T_BLK


def _gate_block(x_ref, w_ref, cw_ref, m8_ref, stats_ref, cnt_ref, gsum_ref):
    i = pl.program_id(0)

    @pl.when(i == 0)
    def _init():
        cnt_ref[...] = jnp.zeros_like(cnt_ref)
        gsum_ref[...] = jnp.zeros_like(gsum_ref)

    x = x_ref[...]
    w = w_ref[...]
    logits = jax.lax.dot_general(
        w, x, (((1,), (1,)), ((), ())), preferred_element_type=jnp.float32
    )
    sub = jax.lax.broadcasted_iota(jnp.int32, (N_EXP, T_BLK), 0)
    m = jnp.max(logits, axis=0, keepdims=True)
    ex = jnp.exp(logits - m)
    gates = ex / jnp.sum(ex, axis=0, keepdims=True)
    gmax = jnp.max(gates, axis=0, keepdims=True)
    eidx = jnp.min(jnp.where(gates == gmax, sub, N_EXP), axis=0, keepdims=True)
    onehot = (sub == eidx).astype(jnp.float32)

    r = jax.lax.broadcasted_iota(jnp.int32, (T_BLK, T_BLK), 0)
    c = jax.lax.broadcasted_iota(jnp.int32, (T_BLK, T_BLK), 1)
    tri = (r <= c).astype(jnp.float32)
    cum = jnp.dot(onehot, tri, preferred_element_type=jnp.float32)

    prev = cnt_ref[...]
    pos = jnp.sum((cum - 1.0 + prev) * onehot, axis=0, keepdims=True)
    pos = pos.astype(jnp.int32)
    keep = pos < CAP
    flat = jnp.where(keep, eidx * CAP + pos, -1)

    j2 = jax.lax.broadcasted_iota(jnp.int32, (N_EXP * CAP, T_BLK), 0)
    hit = j2 == flat
    cw_ref[...] = jnp.where(hit, gmax, 0.0)
    m8_ref[...] = hit.astype(jnp.int8)

    cnt_ref[...] = prev + cum[:, T_BLK - 1 : T_BLK]
    gsum_ref[...] = gsum_ref[...] + jnp.sum(gates, axis=1, keepdims=True)

    @pl.when(i == GRID - 1)
    def _fin():
        cnts = cnt_ref[...]
        gs = gsum_ref[...]
        laux = jnp.sum(cnts * gs) * jnp.float32(N_EXP / (N_TOK * N_TOK))
        lane = jax.lax.broadcasted_iota(jnp.int32, (N_EXP, 8), 1)
        stats_ref[...] = jnp.where(
            lane == 0,
            jnp.broadcast_to(cnts, (N_EXP, 8)),
            jnp.where(lane == 1, jnp.broadcast_to(gs, (N_EXP, 8)), laux),
        )


def _run_gate(x, W):
    return pl.pallas_call(
        _gate_block,
        grid=(GRID,),
        in_specs=[
            pl.BlockSpec((T_BLK, D_MODEL), lambda i: (i, 0)),
            pl.BlockSpec((N_EXP, D_MODEL), lambda i: (0, 0)),
        ],
        out_specs=[
            pl.BlockSpec((N_EXP * CAP, T_BLK), lambda i: (0, i)),
            pl.BlockSpec((N_EXP * CAP, T_BLK), lambda i: (0, i)),
            pl.BlockSpec((N_EXP, 8), lambda i: (0, 0)),
        ],
        out_shape=[
            jax.ShapeDtypeStruct((N_EXP * CAP, N_TOK), jnp.float32),
            jax.ShapeDtypeStruct((N_EXP * CAP, N_TOK), jnp.int8),
            jax.ShapeDtypeStruct((N_EXP, 8), jnp.float32),
        ],
        scratch_shapes=[
            pltpu.VMEM((N_EXP, 1), jnp.float32),
            pltpu.VMEM((N_EXP, 1), jnp.float32),
        ],
        compiler_params=pltpu.CompilerParams(
            dimension_semantics=("arbitrary",),
        ),
    )(x, W)


def _kernel_impl(x, W):
    cw2, m8, stats = _run_gate(x, W)
    l_aux = stats[0, 2]
    exp_counts = stats[:N_EXP, 0].astype(jnp.int32)
    cw3 = cw2.reshape(N_EXP, CAP, N_TOK)
    m3 = m8.reshape(N_EXP, CAP, N_TOK)
    combine_weights = jnp.transpose(cw3, (2, 0, 1))
    dispatch_mask = jnp.transpose(m3, (2, 0, 1)).astype(jnp.bool_)
    return (l_aux, combine_weights, dispatch_mask, exp_counts)


kernel = jax.jit(_kernel_impl)
